# Initial kernel scaffold; baseline (speedup 1.0000x reference)
import jax
import jax.numpy as jnp
from jax.experimental import pallas as pl


def kernel(h, p, e, snorm_n, edge_index, W_pre_h, b_pre_h, W_pre_p, b_pre_p, W_post_h, b_post_h, W_post_p, b_post_p, W_mix_h, b_mix_h, W_mix_p, b_mix_p):
    raise NotImplementedError("write your pallas kernel here")


# speedup vs baseline: 1.1238x; 1.1238x over previous
import functools

import jax
import jax.numpy as jnp
from jax.experimental import pallas as pl

N = 10000
E = 160000
D = 128
EDGE_DIM = 16
AVG_D_LOG = 2.8332133
EPS = 1e-5

ROWS = 1024
NPAD = 10240


def _node_stage(h_ref, p_ref, snorm_ref, deg_ref,
                bh_ref, sum_h_ref, sq_h_ref, mx_h_ref, mn_h_ref,
                bp_ref, sum_p_ref, sq_p_ref, mx_p_ref, mn_p_ref,
                w0h_ref, waggh_ref, wmixh_ref, bposth_ref, bmixh_ref,
                w0p_ref, waggp_ref, wmixp_ref, bpostp_ref, bmixp_ref,
                outh_ref, outp_ref):
    h = h_ref[...]
    p = p_ref[...]
    snorm = snorm_ref[...]
    deg = deg_ref[...]
    degc = jnp.maximum(deg, 1.0)
    has = deg > 0.0
    logd = jnp.log(degc + 1.0)
    s1 = logd * (1.0 / AVG_D_LOG)
    s2 = AVG_D_LOG / logd
    h2 = jnp.concatenate([h, p], axis=1)

    def stats(b, s, q, mx, mn):
        mean_g = s / degc
        msq_g = q / degc
        mean = jnp.where(has, mean_g + b, 0.0)
        msq = jnp.where(has, msq_g + 2.0 * b * mean_g + b * b, 0.0)
        mxo = jnp.where(has, mx + b, 0.0)
        mno = jnp.where(has, mn + b, 0.0)
        var = jnp.maximum(msq - mean * mean, 0.0)
        std = jnp.sqrt(var + EPS)
        return jnp.concatenate([mean, mxo, mno, std], axis=1)

    agg_h = stats(bh_ref[...], sum_h_ref[...], sq_h_ref[...], mx_h_ref[...], mn_h_ref[...])
    agg_p = stats(bp_ref[...], sum_p_ref[...], sq_p_ref[...], mx_p_ref[...], mn_p_ref[...])

    xh = jnp.dot(agg_h, waggh_ref[...], preferred_element_type=jnp.float32)
    h_t = (jnp.dot(h2, w0h_ref[...], preferred_element_type=jnp.float32)
           + xh[:, 0:D] + s1 * xh[:, D:2 * D] + s2 * xh[:, 2 * D:3 * D]
           + bposth_ref[...])
    h_t = h_t * snorm
    hm = jnp.dot(h_t, wmixh_ref[...], preferred_element_type=jnp.float32) + bmixh_ref[...]
    outh_ref[...] = h + jnp.where(hm > 0, hm, 0.01 * hm)

    xp = jnp.dot(agg_p, waggp_ref[...], preferred_element_type=jnp.float32)
    p_t = (jnp.dot(p, w0p_ref[...], preferred_element_type=jnp.float32)
           + xp[:, 0:D] + s1 * xp[:, D:2 * D] + s2 * xp[:, 2 * D:3 * D]
           + bpostp_ref[...])
    pm = jnp.dot(p_t, wmixp_ref[...], preferred_element_type=jnp.float32) + bmixp_ref[...]
    outp_ref[...] = p + jnp.tanh(pm)


def _row_spec(cols):
    return pl.BlockSpec((ROWS, cols), lambda i: (i, 0))


def _full_spec(r, c):
    return pl.BlockSpec((r, c), lambda i: (0, 0))


@functools.partial(jax.jit, static_argnames=())
def _node_stage_call(h, p, snorm, deg,
                     bh, sum_h, sq_h, mx_h, mn_h,
                     bp, sum_p, sq_p, mx_p, mn_p,
                     w0h, waggh, wmixh, bposth, bmixh,
                     w0p, waggp, wmixp, bpostp, bmixp):
    def padrows(x):
        return jnp.pad(x, ((0, NPAD - N), (0, 0)))

    row_args = [padrows(x) for x in
                (h, p, snorm, deg, bh, sum_h, sq_h, mx_h, mn_h,
                 bp, sum_p, sq_p, mx_p, mn_p)]
    in_specs = ([_row_spec(D), _row_spec(D), _row_spec(1), _row_spec(1)]
                + [_row_spec(D)] * 10
                + [_full_spec(2 * D, D), _full_spec(4 * D, 3 * D), _full_spec(D, D),
                   _full_spec(1, D), _full_spec(1, D),
                   _full_spec(D, D), _full_spec(4 * D, 3 * D), _full_spec(D, D),
                   _full_spec(1, D), _full_spec(1, D)])
    outh, outp = pl.pallas_call(
        _node_stage,
        grid=(NPAD // ROWS,),
        in_specs=in_specs,
        out_specs=[_row_spec(D), _row_spec(D)],
        out_shape=[jax.ShapeDtypeStruct((NPAD, D), jnp.float32)] * 2,
    )(*row_args, w0h, waggh, wmixh, bposth, bmixh,
      w0p, waggp, wmixp, bpostp, bmixp)
    return outh[:N], outp[:N]


def kernel(h, p, e, snorm_n, edge_index,
           W_pre_h, b_pre_h, W_pre_p, b_pre_p,
           W_post_h, b_post_h, W_post_p, b_post_p,
           W_mix_h, b_mix_h, W_mix_p, b_mix_p):
    src = edge_index[0]
    dst = edge_index[1]
    h2 = jnp.concatenate([h, p], axis=-1)

    A_h = h2 @ W_pre_h[: 2 * D]
    B_h = h2 @ W_pre_h[2 * D: 4 * D]
    C_h = e @ W_pre_h[4 * D:] + b_pre_h
    A_p = p @ W_pre_p[:D]
    B_p = p @ W_pre_p[D: 2 * D]
    C_p = e @ W_pre_p[2 * D:] + b_pre_p

    g_h = A_h[src] + C_h
    g_p = A_p[src] + C_p

    deg = jax.ops.segment_sum(jnp.ones((E,), jnp.float32), dst, num_segments=N)

    def seg_stats(g):
        s = jax.ops.segment_sum(g, dst, num_segments=N)
        q = jax.ops.segment_sum(g * g, dst, num_segments=N)
        mx = jax.ops.segment_max(g, dst, num_segments=N)
        mn = jax.ops.segment_min(g, dst, num_segments=N)
        return s, q, mx, mn

    sum_h, sq_h, mx_h, mn_h = seg_stats(g_h)
    sum_p, sq_p, mx_p, mn_p = seg_stats(g_p)

    w0h = W_post_h[: 2 * D]
    waggh = jnp.concatenate([W_post_h[2 * D + i * 4 * D: 2 * D + (i + 1) * 4 * D]
                             for i in range(3)], axis=1)
    w0p = W_post_p[:D]
    waggp = jnp.concatenate([W_post_p[D + i * 4 * D: D + (i + 1) * 4 * D]
                             for i in range(3)], axis=1)

    return _node_stage_call(
        h, p, snorm_n, deg[:, None],
        B_h, sum_h, sq_h, mx_h, mn_h,
        B_p, sum_p, sq_p, mx_p, mn_p,
        w0h, waggh, W_mix_h, b_post_h[None, :], b_mix_h[None, :],
        w0p, waggp, W_mix_p, b_post_p[None, :], b_mix_p[None, :])


# speedup vs baseline: 1.1721x; 1.0430x over previous
import functools

import jax
import jax.numpy as jnp
from jax import lax
from jax.experimental import pallas as pl
from jax.experimental.pallas import tpu as pltpu
from jax.experimental.pallas import tpu_sc as plsc

N = 10000
E = 160000
D = 128
EDGE_DIM = 16
AVG_D_LOG = 2.8332133
EPS = 1e-5

ROWS = 1024
NPAD = 10240

NC = 2
NS = 16
NW = NC * NS
RPT = 2
NRANGE = NW * RPT
RNODES = 160
AROWS = 168
CH = 2000
NCHUNK = E // CH
VPC = CH // 16


def _sc_stats_call(dst, src, a_h, c_h, a_p, c_p):
    mesh = plsc.VectorSubcoreMesh(core_axis_name="c", subcore_axis_name="s")

    @functools.partial(
        pl.kernel,
        out_type=[jax.ShapeDtypeStruct((2, 4, NRANGE, AROWS, D), jnp.float32),
                  jax.ShapeDtypeStruct((NRANGE, AROWS, 16), jnp.float32)],
        mesh=mesh,
        compiler_params=pltpu.CompilerParams(needs_layout_passes=False),
        scratch_types=[
            pltpu.VMEM((AROWS, D), jnp.float32),
            pltpu.VMEM((AROWS, D), jnp.float32),
            pltpu.VMEM((AROWS, D), jnp.float32),
            pltpu.VMEM((AROWS, D), jnp.float32),
            pltpu.VMEM((AROWS, 16), jnp.float32),
            pltpu.VMEM((CH,), jnp.int32),
            pltpu.VMEM((CH,), jnp.int32),
            pltpu.VMEM((CH + 16,), jnp.int32),
            pltpu.VMEM((CH + 16,), jnp.int32),
            pltpu.VMEM((CH + 16,), jnp.int32),
            pltpu.VMEM((16, D), jnp.float32),
            pltpu.VMEM((16, D), jnp.float32),
            pltpu.SemaphoreType.DMA,
            pltpu.SemaphoreType.DMA,
        ],
    )
    def k(dst_hbm, src_hbm, ah_hbm, ch_hbm, ap_hbm, cp_hbm,
          out_hbm, deg_hbm,
          acc_sum, acc_sq, acc_mx, acc_mn, acc_deg,
          dst_v, src_v, ids_m, src_m, loc_m, a_buf, c_buf, sem1, sem2):
        wid = lax.axis_index("s") * NC + lax.axis_index("c")

        for f in range(2):
            at_hbm = ah_hbm if f == 0 else ap_hbm
            ct_hbm = ch_hbm if f == 0 else cp_hbm

            def round_body(rnd, _, at_hbm=at_hbm, ct_hbm=ct_hbm, f=f):
                rr = wid * RPT + rnd
                base = rr * RNODES

                def init_body(i, _):
                    zero16 = jnp.zeros((16,), jnp.float32)
                    ninf16 = jnp.full((16,), -jnp.inf, jnp.float32)
                    pinf16 = jnp.full((16,), jnp.inf, jnp.float32)
                    for s in range(8):
                        sl = pl.ds(s * 16, 16)
                        acc_sum[i, sl] = zero16
                        acc_sq[i, sl] = zero16
                        acc_mx[i, sl] = ninf16
                        acc_mn[i, sl] = pinf16
                    if f == 0:
                        acc_deg[i, :] = zero16
                    return 0
                lax.fori_loop(0, AROWS, init_body, 0)

                def chunk_body(c, _):
                    cbase = c * CH
                    pltpu.sync_copy(dst_hbm.at[pl.ds(cbase, CH)], dst_v)
                    pltpu.sync_copy(src_hbm.at[pl.ds(cbase, CH)], src_v)

                    def scan_body(i, nm):
                        lanes = lax.iota(jnp.int32, 16)
                        dv = dst_v[pl.ds(i * 16, 16)]
                        sv = src_v[pl.ds(i * 16, 16)]
                        lv = dv - jnp.full((16,), base, jnp.int32)
                        m = (plsc.bitcast(lv, jnp.uint32)
                             < jnp.full((16,), RNODES, jnp.uint32))
                        csum = plsc.cumsum(jnp.ones((16,), jnp.int32), mask=m)
                        pos = jnp.full((16,), nm - 1, jnp.int32) + csum
                        ids = jnp.full((16,), cbase + i * 16, jnp.int32) + lanes
                        plsc.store_scatter(ids_m, [pos], ids, mask=m)
                        plsc.store_scatter(src_m, [pos], sv, mask=m)
                        plsc.store_scatter(loc_m, [pos], lv, mask=m)
                        cnt = plsc.all_reduce_population_count(m)
                        if cnt.shape:
                            cnt = jnp.max(cnt)
                        return nm + cnt
                    nm = lax.fori_loop(0, VPC, scan_body, 0)

                    ids_m[pl.ds(nm, 16)] = jnp.zeros((16,), jnp.int32)
                    src_m[pl.ds(nm, 16)] = jnp.zeros((16,), jnp.int32)
                    loc_m[pl.ds(nm, 16)] = jnp.full((16,), RNODES, jnp.int32)
                    ngroups = (nm + 15) // 16

                    def group_body(gi, _):
                        gb = gi * 16
                        cp1 = pltpu.async_copy(
                            at_hbm.at[src_m.at[pl.ds(gb, 16)]], a_buf, sem1)
                        cp2 = pltpu.async_copy(
                            ct_hbm.at[ids_m.at[pl.ds(gb, 16)]], c_buf, sem2)
                        cp1.wait()
                        cp2.wait()
                        lanes = lax.iota(jnp.int32, 16)
                        onehot0 = jnp.where(lanes == jnp.zeros((16,), jnp.int32),
                                            jnp.ones((16,), jnp.float32),
                                            jnp.zeros((16,), jnp.float32))
                        lv = loc_m[pl.ds(gb, 16)]
                        for j in range(16):
                            loc = jnp.max(jnp.where(lanes == jnp.full((16,), j, jnp.int32),
                                                    lv, jnp.zeros((16,), jnp.int32)))
                            if f == 0:
                                plsc.addupdate(acc_deg.at[loc, :], onehot0)
                            for s in range(8):
                                sl = pl.ds(s * 16, 16)
                                g = a_buf[j, sl] + c_buf[j, sl]
                                plsc.addupdate(acc_sum.at[loc, sl], g)
                                plsc.addupdate(acc_sq.at[loc, sl], g * g)
                                acc_mx[loc, sl] = jnp.maximum(acc_mx[loc, sl], g)
                                acc_mn[loc, sl] = jnp.minimum(acc_mn[loc, sl], g)
                        return 0
                    lax.fori_loop(0, ngroups, group_body, 0)
                    return 0
                lax.fori_loop(0, NCHUNK, chunk_body, 0)

                pltpu.sync_copy(acc_sum, out_hbm.at[f, 0, rr])
                pltpu.sync_copy(acc_sq, out_hbm.at[f, 1, rr])
                pltpu.sync_copy(acc_mx, out_hbm.at[f, 2, rr])
                pltpu.sync_copy(acc_mn, out_hbm.at[f, 3, rr])
                if f == 0:
                    pltpu.sync_copy(acc_deg, deg_hbm.at[rr])
                return 0
            lax.fori_loop(0, RPT, round_body, 0)

    return k(dst, src, a_h, c_h, a_p, c_p)


def _pre_node(h_ref, p_ref, wh_ref, wp_ref, ah_ref, bh_ref, ap_ref, bp_ref):
    h2 = jnp.concatenate([h_ref[...], p_ref[...]], axis=1)
    x = jnp.dot(h2, wh_ref[...], preferred_element_type=jnp.float32)
    ah_ref[...] = x[:, :D]
    bh_ref[...] = x[:, D:]
    y = jnp.dot(p_ref[...], wp_ref[...], preferred_element_type=jnp.float32)
    ap_ref[...] = y[:, :D]
    bp_ref[...] = y[:, D:]


def _pre_edge(e_ref, weh_ref, beh_ref, wep_ref, bep_ref, ch_ref, cp_ref):
    e = e_ref[...]
    ch_ref[...] = jnp.dot(e, weh_ref[...], preferred_element_type=jnp.float32) + beh_ref[...]
    cp_ref[...] = jnp.dot(e, wep_ref[...], preferred_element_type=jnp.float32) + bep_ref[...]


def _node_stage(h_ref, p_ref, snorm_ref, deg_ref,
                bh_ref, sum_h_ref, sq_h_ref, mx_h_ref, mn_h_ref,
                bp_ref, sum_p_ref, sq_p_ref, mx_p_ref, mn_p_ref,
                w0h_ref, waggh_ref, wmixh_ref, bposth_ref, bmixh_ref,
                w0p_ref, waggp_ref, wmixp_ref, bpostp_ref, bmixp_ref,
                outh_ref, outp_ref):
    h = h_ref[...]
    p = p_ref[...]
    snorm = snorm_ref[...]
    deg = deg_ref[...]
    degc = jnp.maximum(deg, 1.0)
    has = deg > 0.0
    logd = jnp.log(degc + 1.0)
    s1 = logd * (1.0 / AVG_D_LOG)
    s2 = AVG_D_LOG / logd
    h2 = jnp.concatenate([h, p], axis=1)

    def stats(b, s, q, mx, mn):
        mean_g = s / degc
        msq_g = q / degc
        mean = jnp.where(has, mean_g + b, 0.0)
        msq = jnp.where(has, msq_g + 2.0 * b * mean_g + b * b, 0.0)
        mxo = jnp.where(has, mx + b, 0.0)
        mno = jnp.where(has, mn + b, 0.0)
        var = jnp.maximum(msq - mean * mean, 0.0)
        std = jnp.sqrt(var + EPS)
        return jnp.concatenate([mean, mxo, mno, std], axis=1)

    agg_h = stats(bh_ref[...], sum_h_ref[...], sq_h_ref[...], mx_h_ref[...], mn_h_ref[...])
    agg_p = stats(bp_ref[...], sum_p_ref[...], sq_p_ref[...], mx_p_ref[...], mn_p_ref[...])

    xh = jnp.dot(agg_h, waggh_ref[...], preferred_element_type=jnp.float32)
    h_t = (jnp.dot(h2, w0h_ref[...], preferred_element_type=jnp.float32)
           + xh[:, 0:D] + s1 * xh[:, D:2 * D] + s2 * xh[:, 2 * D:3 * D]
           + bposth_ref[...])
    h_t = h_t * snorm
    hm = jnp.dot(h_t, wmixh_ref[...], preferred_element_type=jnp.float32) + bmixh_ref[...]
    outh_ref[...] = h + jnp.where(hm > 0, hm, 0.01 * hm)

    xp = jnp.dot(agg_p, waggp_ref[...], preferred_element_type=jnp.float32)
    p_t = (jnp.dot(p, w0p_ref[...], preferred_element_type=jnp.float32)
           + xp[:, 0:D] + s1 * xp[:, D:2 * D] + s2 * xp[:, 2 * D:3 * D]
           + bpostp_ref[...])
    pm = jnp.dot(p_t, wmixp_ref[...], preferred_element_type=jnp.float32) + bmixp_ref[...]
    outp_ref[...] = p + jnp.tanh(pm)


def _row_spec(cols, rows=ROWS):
    return pl.BlockSpec((rows, cols), lambda i: (i, 0))


def _full_spec(r, c):
    return pl.BlockSpec((r, c), lambda i: (0, 0))


def kernel(h, p, e, snorm_n, edge_index,
           W_pre_h, b_pre_h, W_pre_p, b_pre_p,
           W_post_h, b_post_h, W_post_p, b_post_p,
           W_mix_h, b_mix_h, W_mix_p, b_mix_p):
    src = edge_index[0]
    dst = edge_index[1]

    def padrows(x):
        return jnp.pad(x, ((0, NPAD - x.shape[0]), (0, 0)))

    hp = padrows(h)
    pp = padrows(p)

    wh_cat = jnp.concatenate([W_pre_h[:2 * D], W_pre_h[2 * D:4 * D]], axis=1)
    wp_cat = jnp.concatenate([W_pre_p[:D], W_pre_p[D:2 * D]], axis=1)

    a_h, b_h, a_p, b_p = pl.pallas_call(
        _pre_node,
        grid=(NPAD // ROWS,),
        in_specs=[_row_spec(D), _row_spec(D),
                  _full_spec(2 * D, 2 * D), _full_spec(D, 2 * D)],
        out_specs=[_row_spec(D)] * 4,
        out_shape=[jax.ShapeDtypeStruct((NPAD, D), jnp.float32)] * 4,
    )(hp, pp, wh_cat, wp_cat)

    EROWS = 4000
    c_h, c_p = pl.pallas_call(
        _pre_edge,
        grid=(E // EROWS,),
        in_specs=[_row_spec(EDGE_DIM, EROWS),
                  _full_spec(EDGE_DIM, D), _full_spec(1, D),
                  _full_spec(EDGE_DIM, D), _full_spec(1, D)],
        out_specs=[_row_spec(D, EROWS)] * 2,
        out_shape=[jax.ShapeDtypeStruct((E, D), jnp.float32)] * 2,
    )(e, W_pre_h[4 * D:], b_pre_h[None, :], W_pre_p[2 * D:], b_pre_p[None, :])

    stats, deg_raw = _sc_stats_call(dst, src, a_h, c_h, a_p, c_p)

    st = stats[:, :, :, :RNODES, :].reshape(2, 4, NPAD, D)
    deg = deg_raw[:, :RNODES, 0].reshape(NPAD, 1)

    w0h = W_post_h[:2 * D]
    waggh = jnp.concatenate([W_post_h[2 * D + i * 4 * D: 2 * D + (i + 1) * 4 * D]
                             for i in range(3)], axis=1)
    w0p = W_post_p[:D]
    waggp = jnp.concatenate([W_post_p[D + i * 4 * D: D + (i + 1) * 4 * D]
                             for i in range(3)], axis=1)

    in_specs = ([_row_spec(D), _row_spec(D), _row_spec(1), _row_spec(1)]
                + [_row_spec(D)] * 10
                + [_full_spec(2 * D, D), _full_spec(4 * D, 3 * D), _full_spec(D, D),
                   _full_spec(1, D), _full_spec(1, D),
                   _full_spec(D, D), _full_spec(4 * D, 3 * D), _full_spec(D, D),
                   _full_spec(1, D), _full_spec(1, D)])
    outh, outp = pl.pallas_call(
        _node_stage,
        grid=(NPAD // ROWS,),
        in_specs=in_specs,
        out_specs=[_row_spec(D), _row_spec(D)],
        out_shape=[jax.ShapeDtypeStruct((NPAD, D), jnp.float32)] * 2,
    )(hp, pp, padrows(snorm_n), deg,
      b_h, st[0, 0], st[0, 1], st[0, 2], st[0, 3],
      b_p, st[1, 0], st[1, 1], st[1, 2], st[1, 3],
      w0h, waggh, W_mix_h, b_post_h[None, :], b_mix_h[None, :],
      w0p, waggp, W_mix_p, b_post_p[None, :], b_mix_p[None, :])
    return outh[:N], outp[:N]
